# Initial kernel scaffold; baseline (speedup 1.0000x reference)
import jax
import jax.numpy as jnp
from jax.experimental import pallas as pl


def kernel(node_features, edge_features, Esrc, Etgt, batch, W_in, b_in, ee_W1, ee_b1, ee_W2, ee_b2, gru_Wih, gru_Whh, gru_bih, gru_bhh, lstm_Wih, lstm_Whh, lstm_bih, lstm_bhh, W_out, b_out):
    raise NotImplementedError("write your pallas kernel here")


# speedup vs baseline: 5.3600x; 5.3600x over previous
import functools

import jax
import jax.numpy as jnp
import numpy as np
from jax import lax
from jax.experimental import pallas as pl
from jax.experimental.pallas import tpu as pltpu
from jax.experimental.pallas import tpu_sc as plsc

N = 10000
E = 160000
F_NODE = 128
F_EDGE = 16
H = 16
T = 3
STEPS = 12
B = 64

NC = 2
NS = 16
NW = NC * NS
EW = E // NW
CH = 128
NFULL = EW // CH
TAIL = EW - NFULL * CH
NPS = N // NS


def _sc_mesh():
    return plsc.VectorSubcoreMesh(
        core_axis_name="c", subcore_axis_name="s", num_cores=NC, num_subcores=NS
    )


def _gather_body(h_hbm, idx_hbm, out_hbm, idx_v, rows_v, idxt_v, rowst_v, sem):
    wid = lax.axis_index("s") * NC + lax.axis_index("c")
    base = pl.multiple_of(wid * EW, 8)

    def body(j, _):
        off = pl.multiple_of(base + j * CH, 8)
        pltpu.sync_copy(idx_hbm.at[pl.ds(off, CH)], idx_v)
        pltpu.async_copy(h_hbm.at[idx_v], rows_v, sem).wait()
        pltpu.sync_copy(rows_v, out_hbm.at[pl.ds(off, CH)])
        return 0

    lax.fori_loop(0, NFULL, body, 0)
    off = pl.multiple_of(base + NFULL * CH, 8)
    pltpu.sync_copy(idx_hbm.at[pl.ds(off, TAIL)], idxt_v)
    pltpu.async_copy(h_hbm.at[idxt_v], rowst_v, sem).wait()
    pltpu.sync_copy(rowst_v, out_hbm.at[pl.ds(off, TAIL)])


def _sc_gather(h, esrc):
    k = pl.kernel(
        _gather_body,
        out_type=jax.ShapeDtypeStruct((E, H), jnp.float32),
        mesh=_sc_mesh(),
        scratch_types=[
            pltpu.VMEM((CH,), jnp.int32),
            pltpu.VMEM((CH, H), jnp.float32),
            pltpu.VMEM((TAIL,), jnp.int32),
            pltpu.VMEM((TAIL, H), jnp.float32),
            pltpu.SemaphoreType.DMA,
        ],
        compiler_params=pltpu.CompilerParams(use_tc_tiling_on_sc=False),
    )
    return k(h, esrc)


def _scatter_body(me_hbm, idx_hbm, out_hbm, idx_v, rows_v, idxt_v, rowst_v,
                  buf_v, acc_sh):
    cid = lax.axis_index("c")
    sid = lax.axis_index("s")
    wid = sid * NC + cid
    base = pl.multiple_of(wid * EW, 8)

    def zbody(i, _):
        buf_v[i, :] = jnp.zeros((H,), jnp.float32)
        return 0

    lax.fori_loop(0, NPS, zbody, 0)
    pltpu.sync_copy(buf_v, acc_sh.at[pl.ds(sid * NPS, NPS)])
    plsc.subcore_barrier()

    def body(j, _):
        off = pl.multiple_of(base + j * CH, 8)
        pltpu.sync_copy(idx_hbm.at[pl.ds(off, CH)], idx_v)
        pltpu.sync_copy(me_hbm.at[pl.ds(off, CH)], rows_v)
        pltpu.sync_copy(rows_v, acc_sh.at[idx_v], add=True)
        return 0

    lax.fori_loop(0, NFULL, body, 0)
    off = pl.multiple_of(base + NFULL * CH, 8)
    pltpu.sync_copy(idx_hbm.at[pl.ds(off, TAIL)], idxt_v)
    pltpu.sync_copy(me_hbm.at[pl.ds(off, TAIL)], rowst_v)
    pltpu.sync_copy(rowst_v, acc_sh.at[idxt_v], add=True)
    plsc.subcore_barrier()

    pltpu.sync_copy(acc_sh.at[pl.ds(sid * NPS, NPS)], buf_v)
    pltpu.sync_copy(buf_v, out_hbm.at[cid].at[pl.ds(sid * NPS, NPS)])


def _sc_scatter(m_e, etgt):
    k = pl.kernel(
        _scatter_body,
        out_type=jax.ShapeDtypeStruct((NC, N, H), jnp.float32),
        mesh=_sc_mesh(),
        scratch_types=[
            pltpu.VMEM((CH,), jnp.int32),
            pltpu.VMEM((CH, H), jnp.float32),
            pltpu.VMEM((TAIL,), jnp.int32),
            pltpu.VMEM((TAIL, H), jnp.float32),
            pltpu.VMEM((NPS, H), jnp.float32),
            pltpu.VMEM_SHARED((N, H), jnp.float32),
        ],
        compiler_params=pltpu.CompilerParams(use_tc_tiling_on_sc=False),
    )
    return k(m_e, etgt)


def _encoder_kernel(ef_ref, w1t_ref, b1_ref, out_ref):
    x = jnp.dot(ef_ref[...], w1t_ref[...], preferred_element_type=jnp.float32)
    out_ref[...] = jnp.maximum(x + b1_ref[...], 0.0)


def _proj_kernel(nf_ref, wt_ref, b_ref, out_ref):
    out_ref[...] = (
        jnp.dot(nf_ref[...], wt_ref[...], preferred_element_type=jnp.float32)
        + b_ref[...]
    )


def _me_kernel(hs_ref, eh_ref, vall_ref, b2v_ref, r1_ref, s_ref, out_ref):
    a = (
        jnp.dot(eh_ref[...], vall_ref[...], preferred_element_type=jnp.float32)
        + b2v_ref[...]
    )
    hr = jnp.dot(hs_ref[...], r1_ref[...], preferred_element_type=jnp.float32)
    out_ref[...] = jnp.dot(hr * a, s_ref[...], preferred_element_type=jnp.float32)


def _gru_kernel(h_ref, m2_ref, wih_ref, whh_ref, bih_ref, bhh_ref, out_ref):
    m = m2_ref[0] + m2_ref[1]
    gi = jnp.dot(m, wih_ref[...], preferred_element_type=jnp.float32) + bih_ref[...]
    gh = (
        jnp.dot(h_ref[...], whh_ref[...], preferred_element_type=jnp.float32)
        + bhh_ref[...]
    )
    r = jax.nn.sigmoid(gi[:, 0:H] + gh[:, 0:H])
    z = jax.nn.sigmoid(gi[:, H:2 * H] + gh[:, H:2 * H])
    n = jnp.tanh(gi[:, 2 * H:] + r * gh[:, 2 * H:])
    out_ref[...] = (1.0 - z) * n + z * h_ref[...]


def _set2set_kernel(h_ref, b2d_ref, bt_ref, wiht_ref, whht_ref, bi_ref, bh_ref,
                    woutt_ref, bout_ref, out_ref):
    h = h_ref[...]
    onehot = jnp.where(
        b2d_ref[...] == lax.broadcasted_iota(jnp.int32, (N, B), 1), 1.0, 0.0
    )
    onehot_t = jnp.where(
        bt_ref[...] == lax.broadcasted_iota(jnp.int32, (B, N), 0), 1.0, 0.0
    )
    qstar = jnp.zeros((B, 2 * H), jnp.float32)
    hx = jnp.zeros((B, H), jnp.float32)
    cx = jnp.zeros((B, H), jnp.float32)
    for _ in range(STEPS):
        gates = (
            jnp.dot(qstar, wiht_ref[...], preferred_element_type=jnp.float32)
            + bi_ref[...]
            + jnp.dot(hx, whht_ref[...], preferred_element_type=jnp.float32)
            + bh_ref[...]
        )
        ig = jax.nn.sigmoid(gates[:, 0:H])
        fg = jax.nn.sigmoid(gates[:, H:2 * H])
        gg = jnp.tanh(gates[:, 2 * H:3 * H])
        og = jax.nn.sigmoid(gates[:, 3 * H:])
        cx = fg * cx + ig * gg
        hx = og * jnp.tanh(cx)
        qb = jnp.dot(onehot, hx, preferred_element_type=jnp.float32)
        e = jnp.sum(h * qb, axis=1, keepdims=True)
        em = jnp.where(onehot > 0.0, e, -1e30)
        maxv = jnp.max(em, axis=0, keepdims=True)
        maxn = jnp.sum(onehot * maxv, axis=1, keepdims=True)
        expv = jnp.exp(e - maxn)
        denom = jnp.dot(onehot_t, expv, preferred_element_type=jnp.float32)
        denn = jnp.dot(onehot, denom, preferred_element_type=jnp.float32)
        a = expv / denn
        rvec = jnp.dot(onehot_t, a * h, preferred_element_type=jnp.float32)
        qstar = jnp.concatenate([hx, rvec], axis=1)
    out_ref[...] = (
        jnp.dot(hx, woutt_ref[...], preferred_element_type=jnp.float32)
        + bout_ref[...]
    )


def _full(shape):
    return pl.BlockSpec(shape, lambda *_: tuple(0 for _ in shape))


def kernel(node_features, edge_features, Esrc, Etgt, batch,
           W_in, b_in, ee_W1, ee_b1, ee_W2, ee_b2,
           gru_Wih, gru_Whh, gru_bih, gru_bhh,
           lstm_Wih, lstm_Whh, lstm_bih, lstm_bhh,
           W_out, b_out):
    f32 = jnp.float32
    Esrc = Esrc.astype(jnp.int32)
    Etgt = Etgt.astype(jnp.int32)

    r1 = jnp.asarray(np.repeat(np.eye(H, dtype=np.float32), H, axis=1))
    s = jnp.asarray(np.tile(np.eye(H, dtype=np.float32), (H, 1)))
    vall = ee_W2.reshape(H, H, H).transpose(2, 1, 0).reshape(H, H * H)
    b2v = ee_b2.reshape(H, H).T.reshape(1, H * H)

    BE2 = 8000
    eh = pl.pallas_call(
        _encoder_kernel,
        out_shape=jax.ShapeDtypeStruct((E, H), f32),
        grid=(E // BE2,),
        in_specs=[
            pl.BlockSpec((BE2, F_EDGE), lambda i: (i, 0)),
            _full((F_EDGE, H)),
            _full((1, H)),
        ],
        out_specs=pl.BlockSpec((BE2, H), lambda i: (i, 0)),
    )(edge_features, ee_W1.T, ee_b1.reshape(1, H))

    h = pl.pallas_call(
        _proj_kernel,
        out_shape=jax.ShapeDtypeStruct((N, H), f32),
        in_specs=[_full((N, F_NODE)), _full((F_NODE, H)), _full((1, H))],
        out_specs=_full((N, H)),
    )(node_features, W_in.T, b_in.reshape(1, H))

    BE = 4000
    me_call = pl.pallas_call(
        _me_kernel,
        out_shape=jax.ShapeDtypeStruct((E, H), f32),
        grid=(E // BE,),
        in_specs=[
            pl.BlockSpec((BE, H), lambda i: (i, 0)),
            pl.BlockSpec((BE, H), lambda i: (i, 0)),
            _full((H, H * H)),
            _full((1, H * H)),
            _full((H, H * H)),
            _full((H * H, H)),
        ],
        out_specs=pl.BlockSpec((BE, H), lambda i: (i, 0)),
    )

    gru_call = pl.pallas_call(
        _gru_kernel,
        out_shape=jax.ShapeDtypeStruct((N, H), f32),
        in_specs=[
            _full((N, H)),
            _full((NC, N, H)),
            _full((H, 3 * H)),
            _full((H, 3 * H)),
            _full((1, 3 * H)),
            _full((1, 3 * H)),
        ],
        out_specs=_full((N, H)),
    )
    wih_t = gru_Wih.T
    whh_t = gru_Whh.T
    bih2 = gru_bih.reshape(1, 3 * H)
    bhh2 = gru_bhh.reshape(1, 3 * H)

    for _ in range(T):
        hs = _sc_gather(h, Esrc)
        m_e = me_call(hs, eh, vall, b2v, r1, s)
        m2 = _sc_scatter(m_e, Etgt)
        h = gru_call(h, m2, wih_t, whh_t, bih2, bhh2)

    out = pl.pallas_call(
        _set2set_kernel,
        out_shape=jax.ShapeDtypeStruct((B, 1), f32),
        in_specs=[
            _full((N, H)),
            _full((N, 1)),
            _full((1, N)),
            _full((2 * H, 4 * H)),
            _full((H, 4 * H)),
            _full((1, 4 * H)),
            _full((1, 4 * H)),
            _full((H, 1)),
            _full((1, 1)),
        ],
        out_specs=_full((B, 1)),
    )(
        h,
        batch.astype(jnp.int32).reshape(N, 1),
        batch.astype(jnp.int32).reshape(1, N),
        lstm_Wih.T,
        lstm_Whh.T,
        lstm_bih.reshape(1, 4 * H),
        lstm_bhh.reshape(1, 4 * H),
        W_out.T,
        b_out.reshape(1, 1),
    )
    return out


# speedup vs baseline: 6.2051x; 1.1577x over previous
import functools

import jax
import jax.numpy as jnp
import numpy as np
from jax import lax
from jax.experimental import pallas as pl
from jax.experimental.pallas import tpu as pltpu
from jax.experimental.pallas import tpu_sc as plsc

N = 10000
E = 160000
F_NODE = 128
F_EDGE = 16
H = 16
T = 3
STEPS = 12
B = 64

NC = 2
NS = 16
NW = NC * NS
EW = E // NW
CH = 128
CHUNKS = 40
EWP = CHUNKS * CH
PAD = EWP - EW
ACC_N = 10240
NPS = N // NS
NPZ = ACC_N // NS


def _sc_mesh():
    return plsc.VectorSubcoreMesh(
        core_axis_name="c", subcore_axis_name="s", num_cores=NC, num_subcores=NS
    )


def _gather_body(h_hbm, idx_hbm, out_hbm, idx_v, rows_v, sem):
    wid = lax.axis_index("s") * NC + lax.axis_index("c")
    pltpu.sync_copy(idx_hbm.at[wid], idx_v)
    descs = [
        pltpu.async_copy(
            h_hbm.at[idx_v.at[j]], rows_v.at[pl.ds(j * CH, CH)], sem
        )
        for j in range(CHUNKS)
    ]
    for d in descs:
        d.wait()
    base = pl.multiple_of(wid * EW, 8)
    pltpu.sync_copy(rows_v.at[pl.ds(0, EW)], out_hbm.at[pl.ds(base, EW)])


def _sc_gather(h, es3):
    k = pl.kernel(
        _gather_body,
        out_type=jax.ShapeDtypeStruct((E, H), jnp.float32),
        mesh=_sc_mesh(),
        scratch_types=[
            pltpu.VMEM((CHUNKS, CH), jnp.int32),
            pltpu.VMEM((EWP, H), jnp.float32),
            pltpu.SemaphoreType.DMA,
        ],
        compiler_params=pltpu.CompilerParams(use_tc_tiling_on_sc=False),
    )
    return k(h, es3)


def _scatter_body(me_hbm, idx_hbm, out_hbm, idx_v, rows_v, buf_v, acc_sh, sem):
    cid = lax.axis_index("c")
    sid = lax.axis_index("s")
    wid = sid * NC + cid

    def zbody(i, _):
        buf_v[i, :] = jnp.zeros((H,), jnp.float32)
        return 0

    lax.fori_loop(0, NPZ, zbody, 0)
    pltpu.sync_copy(buf_v, acc_sh.at[pl.ds(sid * NPZ, NPZ)])
    pltpu.sync_copy(idx_hbm.at[wid], idx_v)
    base = pl.multiple_of(wid * EW, 8)
    pltpu.sync_copy(me_hbm.at[pl.ds(base, EW)], rows_v.at[pl.ds(0, EW)])
    plsc.subcore_barrier()
    descs = [
        pltpu.async_copy(
            rows_v.at[pl.ds(j * CH, CH)], acc_sh.at[idx_v.at[j]], sem, add=True
        )
        for j in range(CHUNKS)
    ]
    for d in descs:
        d.wait()
    plsc.subcore_barrier()

    pltpu.sync_copy(acc_sh.at[pl.ds(sid * NPS, NPS)], buf_v.at[pl.ds(0, NPS)])
    pltpu.sync_copy(buf_v.at[pl.ds(0, NPS)], out_hbm.at[cid].at[pl.ds(sid * NPS, NPS)])


def _sc_scatter(m_e, et3):
    k = pl.kernel(
        _scatter_body,
        out_type=jax.ShapeDtypeStruct((NC, N, H), jnp.float32),
        mesh=_sc_mesh(),
        scratch_types=[
            pltpu.VMEM((CHUNKS, CH), jnp.int32),
            pltpu.VMEM((EWP, H), jnp.float32),
            pltpu.VMEM((NPZ, H), jnp.float32),
            pltpu.VMEM_SHARED((ACC_N, H), jnp.float32),
            pltpu.SemaphoreType.DMA,
        ],
        compiler_params=pltpu.CompilerParams(use_tc_tiling_on_sc=False),
    )
    return k(m_e, et3)


def _encoder_kernel(ef_ref, w1t_ref, b1_ref, out_ref):
    x = jnp.dot(ef_ref[...], w1t_ref[...], preferred_element_type=jnp.float32)
    out_ref[...] = jnp.maximum(x + b1_ref[...], 0.0)


def _proj_kernel(nf_ref, wt_ref, b_ref, out_ref):
    out_ref[...] = (
        jnp.dot(nf_ref[...], wt_ref[...], preferred_element_type=jnp.float32)
        + b_ref[...]
    )


def _me_kernel(hs_ref, eh_ref, vall_ref, b2v_ref, r1_ref, s_ref, out_ref):
    a = (
        jnp.dot(eh_ref[...], vall_ref[...], preferred_element_type=jnp.float32)
        + b2v_ref[...]
    )
    hr = jnp.dot(hs_ref[...], r1_ref[...], preferred_element_type=jnp.float32)
    out_ref[...] = jnp.dot(hr * a, s_ref[...], preferred_element_type=jnp.float32)


def _gru_kernel(h_ref, m2_ref, wih_ref, whh_ref, bih_ref, bhh_ref, out_ref):
    m = m2_ref[0] + m2_ref[1]
    gi = jnp.dot(m, wih_ref[...], preferred_element_type=jnp.float32) + bih_ref[...]
    gh = (
        jnp.dot(h_ref[...], whh_ref[...], preferred_element_type=jnp.float32)
        + bhh_ref[...]
    )
    r = jax.nn.sigmoid(gi[:, 0:H] + gh[:, 0:H])
    z = jax.nn.sigmoid(gi[:, H:2 * H] + gh[:, H:2 * H])
    n = jnp.tanh(gi[:, 2 * H:] + r * gh[:, 2 * H:])
    out_ref[...] = (1.0 - z) * n + z * h_ref[...]


def _set2set_kernel(h_ref, b2d_ref, bt_ref, wiht_ref, whht_ref, bi_ref, bh_ref,
                    woutt_ref, bout_ref, out_ref):
    h = h_ref[...]
    onehot = jnp.where(
        b2d_ref[...] == lax.broadcasted_iota(jnp.int32, (N, B), 1), 1.0, 0.0
    )
    onehot_t = jnp.where(
        bt_ref[...] == lax.broadcasted_iota(jnp.int32, (B, N), 0), 1.0, 0.0
    )
    qstar = jnp.zeros((B, 2 * H), jnp.float32)
    hx = jnp.zeros((B, H), jnp.float32)
    cx = jnp.zeros((B, H), jnp.float32)
    for _ in range(STEPS):
        gates = (
            jnp.dot(qstar, wiht_ref[...], preferred_element_type=jnp.float32)
            + bi_ref[...]
            + jnp.dot(hx, whht_ref[...], preferred_element_type=jnp.float32)
            + bh_ref[...]
        )
        ig = jax.nn.sigmoid(gates[:, 0:H])
        fg = jax.nn.sigmoid(gates[:, H:2 * H])
        gg = jnp.tanh(gates[:, 2 * H:3 * H])
        og = jax.nn.sigmoid(gates[:, 3 * H:])
        cx = fg * cx + ig * gg
        hx = og * jnp.tanh(cx)
        qb = jnp.dot(onehot, hx, preferred_element_type=jnp.float32)
        e = jnp.sum(h * qb, axis=1, keepdims=True)
        em = jnp.where(onehot > 0.0, e, -1e30)
        maxv = jnp.max(em, axis=0, keepdims=True)
        maxn = jnp.sum(onehot * maxv, axis=1, keepdims=True)
        expv = jnp.exp(e - maxn)
        denom = jnp.dot(onehot_t, expv, preferred_element_type=jnp.float32)
        denn = jnp.dot(onehot, denom, preferred_element_type=jnp.float32)
        a = expv / denn
        rvec = jnp.dot(onehot_t, a * h, preferred_element_type=jnp.float32)
        qstar = jnp.concatenate([hx, rvec], axis=1)
    out_ref[...] = (
        jnp.dot(hx, woutt_ref[...], preferred_element_type=jnp.float32)
        + bout_ref[...]
    )


def _full(shape):
    return pl.BlockSpec(shape, lambda *_: tuple(0 for _ in shape))


def kernel(node_features, edge_features, Esrc, Etgt, batch,
           W_in, b_in, ee_W1, ee_b1, ee_W2, ee_b2,
           gru_Wih, gru_Whh, gru_bih, gru_bhh,
           lstm_Wih, lstm_Whh, lstm_bih, lstm_bhh,
           W_out, b_out):
    f32 = jnp.float32
    es3 = jnp.pad(
        Esrc.astype(jnp.int32).reshape(NW, EW), ((0, 0), (0, PAD))
    ).reshape(NW, CHUNKS, CH)
    et3 = jnp.pad(
        Etgt.astype(jnp.int32).reshape(NW, EW), ((0, 0), (0, PAD)),
        constant_values=N,
    ).reshape(NW, CHUNKS, CH)

    r1 = jnp.asarray(np.repeat(np.eye(H, dtype=np.float32), H, axis=1))
    s = jnp.asarray(np.tile(np.eye(H, dtype=np.float32), (H, 1)))
    vall = ee_W2.reshape(H, H, H).transpose(2, 1, 0).reshape(H, H * H)
    b2v = ee_b2.reshape(H, H).T.reshape(1, H * H)

    BE2 = 8000
    eh = pl.pallas_call(
        _encoder_kernel,
        out_shape=jax.ShapeDtypeStruct((E, H), f32),
        grid=(E // BE2,),
        in_specs=[
            pl.BlockSpec((BE2, F_EDGE), lambda i: (i, 0)),
            _full((F_EDGE, H)),
            _full((1, H)),
        ],
        out_specs=pl.BlockSpec((BE2, H), lambda i: (i, 0)),
    )(edge_features, ee_W1.T, ee_b1.reshape(1, H))

    h = pl.pallas_call(
        _proj_kernel,
        out_shape=jax.ShapeDtypeStruct((N, H), f32),
        in_specs=[_full((N, F_NODE)), _full((F_NODE, H)), _full((1, H))],
        out_specs=_full((N, H)),
    )(node_features, W_in.T, b_in.reshape(1, H))

    BE = 4000
    me_call = pl.pallas_call(
        _me_kernel,
        out_shape=jax.ShapeDtypeStruct((E, H), f32),
        grid=(E // BE,),
        in_specs=[
            pl.BlockSpec((BE, H), lambda i: (i, 0)),
            pl.BlockSpec((BE, H), lambda i: (i, 0)),
            _full((H, H * H)),
            _full((1, H * H)),
            _full((H, H * H)),
            _full((H * H, H)),
        ],
        out_specs=pl.BlockSpec((BE, H), lambda i: (i, 0)),
    )

    gru_call = pl.pallas_call(
        _gru_kernel,
        out_shape=jax.ShapeDtypeStruct((N, H), f32),
        in_specs=[
            _full((N, H)),
            _full((NC, N, H)),
            _full((H, 3 * H)),
            _full((H, 3 * H)),
            _full((1, 3 * H)),
            _full((1, 3 * H)),
        ],
        out_specs=_full((N, H)),
    )
    wih_t = gru_Wih.T
    whh_t = gru_Whh.T
    bih2 = gru_bih.reshape(1, 3 * H)
    bhh2 = gru_bhh.reshape(1, 3 * H)

    for _ in range(T):
        hs = _sc_gather(h, es3)
        m_e = me_call(hs, eh, vall, b2v, r1, s)
        m2 = _sc_scatter(m_e, et3)
        h = gru_call(h, m2, wih_t, whh_t, bih2, bhh2)

    out = pl.pallas_call(
        _set2set_kernel,
        out_shape=jax.ShapeDtypeStruct((B, 1), f32),
        in_specs=[
            _full((N, H)),
            _full((N, 1)),
            _full((1, N)),
            _full((2 * H, 4 * H)),
            _full((H, 4 * H)),
            _full((1, 4 * H)),
            _full((1, 4 * H)),
            _full((H, 1)),
            _full((1, 1)),
        ],
        out_specs=_full((B, 1)),
    )(
        h,
        batch.astype(jnp.int32).reshape(N, 1),
        batch.astype(jnp.int32).reshape(1, N),
        lstm_Wih.T,
        lstm_Whh.T,
        lstm_bih.reshape(1, 4 * H),
        lstm_bhh.reshape(1, 4 * H),
        W_out.T,
        b_out.reshape(1, 1),
    )
    return out


# speedup vs baseline: 10.0596x; 1.6212x over previous
import functools

import jax
import jax.numpy as jnp
import numpy as np
from jax import lax
from jax.experimental import pallas as pl
from jax.experimental.pallas import tpu as pltpu
from jax.experimental.pallas import tpu_sc as plsc

N = 10000
E = 160000
F_NODE = 128
F_EDGE = 16
H = 16
T = 3
STEPS = 12
B = 64

NC = 2
NS = 16
NW = NC * NS
EW = E // NW
CH = 128
CHUNKS = 40
EWP = CHUNKS * CH
PAD = EWP - EW
ACC_N = 10240
NPS = N // NS
NPZ = ACC_N // NS
EP = 163840
RP = EP // 8
RME = 512


def _sc_mesh():
    return plsc.VectorSubcoreMesh(
        core_axis_name="c", subcore_axis_name="s", num_cores=NC, num_subcores=NS
    )


def _gather_body(h_hbm, idx_hbm, out_hbm, idx_v, rows_v, sem):
    wid = lax.axis_index("s") * NC + lax.axis_index("c")
    pltpu.sync_copy(idx_hbm.at[wid], idx_v)
    descs = [
        pltpu.async_copy(
            h_hbm.at[idx_v.at[j]], rows_v.at[pl.ds(j * CH, CH)], sem
        )
        for j in range(CHUNKS)
    ]
    for d in descs:
        d.wait()
    base = pl.multiple_of(wid * EW, 8)
    pltpu.sync_copy(rows_v.at[pl.ds(0, EW)], out_hbm.at[pl.ds(base, EW)])


def _sc_gather(h, es3):
    k = pl.kernel(
        _gather_body,
        out_type=jax.ShapeDtypeStruct((EP, H), jnp.float32),
        mesh=_sc_mesh(),
        scratch_types=[
            pltpu.VMEM((CHUNKS, CH), jnp.int32),
            pltpu.VMEM((EWP, H), jnp.float32),
            pltpu.SemaphoreType.DMA,
        ],
        compiler_params=pltpu.CompilerParams(use_tc_tiling_on_sc=False),
    )
    return k(h, es3)


def _scatter_body(me_hbm, idx_hbm, out_hbm, idx_v, rows_v, buf_v, acc_sh, sem):
    cid = lax.axis_index("c")
    sid = lax.axis_index("s")
    wid = sid * NC + cid

    def zbody(i, _):
        buf_v[i, :] = jnp.zeros((H,), jnp.float32)
        return 0

    lax.fori_loop(0, NPZ, zbody, 0)
    pltpu.sync_copy(buf_v, acc_sh.at[pl.ds(sid * NPZ, NPZ)])
    pltpu.sync_copy(idx_hbm.at[wid], idx_v)
    base = pl.multiple_of(wid * EW, 8)
    pltpu.sync_copy(me_hbm.at[pl.ds(base, EW)], rows_v.at[pl.ds(0, EW)])
    plsc.subcore_barrier()
    descs = [
        pltpu.async_copy(
            rows_v.at[pl.ds(j * CH, CH)], acc_sh.at[idx_v.at[j]], sem, add=True
        )
        for j in range(CHUNKS)
    ]
    for d in descs:
        d.wait()
    plsc.subcore_barrier()

    pltpu.sync_copy(acc_sh.at[pl.ds(sid * NPS, NPS)], buf_v.at[pl.ds(0, NPS)])
    pltpu.sync_copy(buf_v.at[pl.ds(0, NPS)], out_hbm.at[cid].at[pl.ds(sid * NPS, NPS)])


def _sc_scatter(m_e, et3):
    k = pl.kernel(
        _scatter_body,
        out_type=jax.ShapeDtypeStruct((NC, N, H), jnp.float32),
        mesh=_sc_mesh(),
        scratch_types=[
            pltpu.VMEM((CHUNKS, CH), jnp.int32),
            pltpu.VMEM((EWP, H), jnp.float32),
            pltpu.VMEM((NPZ, H), jnp.float32),
            pltpu.VMEM_SHARED((ACC_N, H), jnp.float32),
            pltpu.SemaphoreType.DMA,
        ],
        compiler_params=pltpu.CompilerParams(use_tc_tiling_on_sc=False),
    )
    return k(m_e, et3)


def _proj_kernel(nf_ref, wt_ref, b_ref, out_ref):
    out_ref[...] = (
        jnp.dot(nf_ref[...], wt_ref[...], preferred_element_type=jnp.float32)
        + b_ref[...]
    )


def _me_kernel(hs_ref, ef_ref, w1p_ref, b1p_ref, vt_ref, bt_ref, r1t_ref,
               st_ref, out_ref):
    f32 = jnp.float32
    ehp = jnp.maximum(
        jnp.dot(ef_ref[...], w1p_ref[...], preferred_element_type=f32)
        + b1p_ref[...], 0.0)
    ehT = ehp.T.reshape(8, H, RME).transpose(1, 0, 2).reshape(H, 8 * RME)
    hsT = hs_ref[...].T.reshape(8, H, RME).transpose(1, 0, 2).reshape(H, 8 * RME)
    aT = jnp.dot(vt_ref[...], ehT, preferred_element_type=f32) + bt_ref[...]
    hrT = jnp.dot(r1t_ref[...], hsT, preferred_element_type=f32)
    meT = jnp.dot(st_ref[...], hrT * aT, preferred_element_type=f32)
    mePT = meT.reshape(H, 8, RME).transpose(1, 0, 2).reshape(128, RME)
    out_ref[...] = mePT.T


def _gru_kernel(h_ref, m2_ref, wih_ref, whh_ref, bih_ref, bhh_ref, out_ref):
    m = m2_ref[0] + m2_ref[1]
    gi = jnp.dot(m, wih_ref[...], preferred_element_type=jnp.float32) + bih_ref[...]
    gh = (
        jnp.dot(h_ref[...], whh_ref[...], preferred_element_type=jnp.float32)
        + bhh_ref[...]
    )
    r = jax.nn.sigmoid(gi[:, 0:H] + gh[:, 0:H])
    z = jax.nn.sigmoid(gi[:, H:2 * H] + gh[:, H:2 * H])
    n = jnp.tanh(gi[:, 2 * H:] + r * gh[:, 2 * H:])
    out_ref[...] = (1.0 - z) * n + z * h_ref[...]


def _set2set_kernel(h_ref, b2d_ref, bt_ref, wiht_ref, whht_ref, bi_ref, bh_ref,
                    woutt_ref, bout_ref, out_ref):
    h = h_ref[...]
    onehot = jnp.where(
        b2d_ref[...] == lax.broadcasted_iota(jnp.int32, (N, B), 1), 1.0, 0.0
    )
    onehot_t = jnp.where(
        bt_ref[...] == lax.broadcasted_iota(jnp.int32, (B, N), 0), 1.0, 0.0
    )
    qstar = jnp.zeros((B, 2 * H), jnp.float32)
    hx = jnp.zeros((B, H), jnp.float32)
    cx = jnp.zeros((B, H), jnp.float32)
    for _ in range(STEPS):
        gates = (
            jnp.dot(qstar, wiht_ref[...], preferred_element_type=jnp.float32)
            + bi_ref[...]
            + jnp.dot(hx, whht_ref[...], preferred_element_type=jnp.float32)
            + bh_ref[...]
        )
        ig = jax.nn.sigmoid(gates[:, 0:H])
        fg = jax.nn.sigmoid(gates[:, H:2 * H])
        gg = jnp.tanh(gates[:, 2 * H:3 * H])
        og = jax.nn.sigmoid(gates[:, 3 * H:])
        cx = fg * cx + ig * gg
        hx = og * jnp.tanh(cx)
        qb = jnp.dot(onehot, hx, preferred_element_type=jnp.float32)
        e = jnp.sum(h * qb, axis=1, keepdims=True)
        em = jnp.where(onehot > 0.0, e, -1e30)
        maxv = jnp.max(em, axis=0, keepdims=True)
        maxn = jnp.sum(onehot * maxv, axis=1, keepdims=True)
        expv = jnp.exp(e - maxn)
        denom = jnp.dot(onehot_t, expv, preferred_element_type=jnp.float32)
        denn = jnp.dot(onehot, denom, preferred_element_type=jnp.float32)
        a = expv / denn
        rvec = jnp.dot(onehot_t, a * h, preferred_element_type=jnp.float32)
        qstar = jnp.concatenate([hx, rvec], axis=1)
    out_ref[...] = (
        jnp.dot(hx, woutt_ref[...], preferred_element_type=jnp.float32)
        + bout_ref[...]
    )


def _full(shape):
    return pl.BlockSpec(shape, lambda *_: tuple(0 for _ in shape))


def kernel(node_features, edge_features, Esrc, Etgt, batch,
           W_in, b_in, ee_W1, ee_b1, ee_W2, ee_b2,
           gru_Wih, gru_Whh, gru_bih, gru_bhh,
           lstm_Wih, lstm_Whh, lstm_bih, lstm_bhh,
           W_out, b_out):
    f32 = jnp.float32
    es3 = jnp.pad(
        Esrc.astype(jnp.int32).reshape(NW, EW), ((0, 0), (0, PAD))
    ).reshape(NW, CHUNKS, CH)
    et3 = jnp.pad(
        Etgt.astype(jnp.int32).reshape(NW, EW), ((0, 0), (0, PAD)),
        constant_values=N,
    ).reshape(NW, CHUNKS, CH)

    r1t = jnp.asarray(
        np.repeat(np.eye(H, dtype=np.float32), H, axis=1).T
    )
    st = jnp.asarray(
        np.tile(np.eye(H, dtype=np.float32), (H, 1)).T
    )
    vallT = ee_W2.reshape(H, H, H).transpose(1, 0, 2).reshape(H * H, H)
    btT = ee_b2.reshape(H, H).T.reshape(H * H, 1)
    w1p = jnp.kron(jnp.eye(8, dtype=f32), ee_W1.T)
    b1p = jnp.tile(ee_b1, 8).reshape(1, 128)

    ef128 = jnp.pad(edge_features.reshape(E // 8, 128),
                    ((0, (EP - E) // 8), (0, 0)))

    h = pl.pallas_call(
        _proj_kernel,
        out_shape=jax.ShapeDtypeStruct((N, H), f32),
        in_specs=[_full((N, F_NODE)), _full((F_NODE, H)), _full((1, H))],
        out_specs=_full((N, H)),
    )(node_features, W_in.T, b_in.reshape(1, H))

    me_call = pl.pallas_call(
        _me_kernel,
        out_shape=jax.ShapeDtypeStruct((RP, 128), f32),
        grid=(RP // RME,),
        in_specs=[
            pl.BlockSpec((RME, 128), lambda i: (i, 0)),
            pl.BlockSpec((RME, 128), lambda i: (i, 0)),
            _full((128, 128)),
            _full((1, 128)),
            _full((H * H, H)),
            _full((H * H, 1)),
            _full((H * H, H)),
            _full((H, H * H)),
        ],
        out_specs=pl.BlockSpec((RME, 128), lambda i: (i, 0)),
    )

    gru_call = pl.pallas_call(
        _gru_kernel,
        out_shape=jax.ShapeDtypeStruct((N, H), f32),
        in_specs=[
            _full((N, H)),
            _full((NC, N, H)),
            _full((H, 3 * H)),
            _full((H, 3 * H)),
            _full((1, 3 * H)),
            _full((1, 3 * H)),
        ],
        out_specs=_full((N, H)),
    )
    wih_t = gru_Wih.T
    whh_t = gru_Whh.T
    bih2 = gru_bih.reshape(1, 3 * H)
    bhh2 = gru_bhh.reshape(1, 3 * H)

    for _ in range(T):
        hs = _sc_gather(h, es3)
        me128 = me_call(hs.reshape(RP, 128), ef128, w1p, b1p, vallT, btT,
                        r1t, st)
        m2 = _sc_scatter(me128.reshape(EP, H), et3)
        h = gru_call(h, m2, wih_t, whh_t, bih2, bhh2)

    out = pl.pallas_call(
        _set2set_kernel,
        out_shape=jax.ShapeDtypeStruct((B, 1), f32),
        in_specs=[
            _full((N, H)),
            _full((N, 1)),
            _full((1, N)),
            _full((2 * H, 4 * H)),
            _full((H, 4 * H)),
            _full((1, 4 * H)),
            _full((1, 4 * H)),
            _full((H, 1)),
            _full((1, 1)),
        ],
        out_specs=_full((B, 1)),
    )(
        h,
        batch.astype(jnp.int32).reshape(N, 1),
        batch.astype(jnp.int32).reshape(1, N),
        lstm_Wih.T,
        lstm_Whh.T,
        lstm_bih.reshape(1, 4 * H),
        lstm_bhh.reshape(1, 4 * H),
        W_out.T,
        b_out.reshape(1, 1),
    )
    return out


# speedup vs baseline: 11.9198x; 1.1849x over previous
import functools

import jax
import jax.numpy as jnp
import numpy as np
from jax import lax
from jax.experimental import pallas as pl
from jax.experimental.pallas import tpu as pltpu
from jax.experimental.pallas import tpu_sc as plsc

N = 10000
E = 160000
F_NODE = 128
F_EDGE = 16
H = 16
T = 3
STEPS = 12
B = 64

NC = 2
NS = 16
NW = NC * NS
EW = E // NW
CH = 128
CHUNKS = 40
EWP = CHUNKS * CH
PAD = EWP - EW
ACC_N = 10240
NPS = N // NS
NPZ = ACC_N // NS
EP = 163840
RP = EP // 8
RME = 512


def _sc_mesh():
    return plsc.VectorSubcoreMesh(
        core_axis_name="c", subcore_axis_name="s", num_cores=NC, num_subcores=NS
    )


def _gather_body(h_hbm, idx_hbm, out_hbm, idx_v, rows_v, sem):
    wid = lax.axis_index("s") * NC + lax.axis_index("c")
    pltpu.sync_copy(idx_hbm.at[wid], idx_v)
    descs = [
        pltpu.async_copy(
            h_hbm.at[idx_v.at[j]], rows_v.at[pl.ds(j * CH, CH)], sem
        )
        for j in range(CHUNKS)
    ]
    for d in descs:
        d.wait()
    base = pl.multiple_of(wid * EW, 8)
    pltpu.sync_copy(rows_v.at[pl.ds(0, EW)], out_hbm.at[pl.ds(base, EW)])


def _sc_gather(h, es3):
    k = pl.kernel(
        _gather_body,
        out_type=jax.ShapeDtypeStruct((EP, H), jnp.float32),
        mesh=_sc_mesh(),
        scratch_types=[
            pltpu.VMEM((CHUNKS, CH), jnp.int32),
            pltpu.VMEM((EWP, H), jnp.float32),
            pltpu.SemaphoreType.DMA,
        ],
        compiler_params=pltpu.CompilerParams(use_tc_tiling_on_sc=False),
    )
    return k(h, es3)


def _scatter_body(me_hbm, idx_hbm, out_hbm, idx_v, rows_v, buf_v, acc_sh, sem):
    cid = lax.axis_index("c")
    sid = lax.axis_index("s")
    wid = sid * NC + cid

    def zbody(i, _):
        buf_v[i, :] = jnp.zeros((H,), jnp.float32)
        return 0

    lax.fori_loop(0, NPZ, zbody, 0)
    pltpu.sync_copy(buf_v, acc_sh.at[pl.ds(sid * NPZ, NPZ)])
    pltpu.sync_copy(idx_hbm.at[wid], idx_v)
    base = pl.multiple_of(wid * EW, 8)
    pltpu.sync_copy(me_hbm.at[pl.ds(base, EW)], rows_v.at[pl.ds(0, EW)])
    plsc.subcore_barrier()
    descs = [
        pltpu.async_copy(
            rows_v.at[pl.ds(j * CH, CH)], acc_sh.at[idx_v.at[j]], sem, add=True
        )
        for j in range(CHUNKS)
    ]
    for d in descs:
        d.wait()
    plsc.subcore_barrier()

    pltpu.sync_copy(acc_sh.at[pl.ds(sid * NPS, NPS)], buf_v.at[pl.ds(0, NPS)])
    pltpu.sync_copy(buf_v.at[pl.ds(0, NPS)], out_hbm.at[cid].at[pl.ds(sid * NPS, NPS)])


def _sc_scatter(m_e, et3):
    k = pl.kernel(
        _scatter_body,
        out_type=jax.ShapeDtypeStruct((NC, N, H), jnp.float32),
        mesh=_sc_mesh(),
        scratch_types=[
            pltpu.VMEM((CHUNKS, CH), jnp.int32),
            pltpu.VMEM((EWP, H), jnp.float32),
            pltpu.VMEM((NPZ, H), jnp.float32),
            pltpu.VMEM_SHARED((ACC_N, H), jnp.float32),
            pltpu.SemaphoreType.DMA,
        ],
        compiler_params=pltpu.CompilerParams(use_tc_tiling_on_sc=False),
    )
    return k(m_e, et3)


def _proj_kernel(nf_ref, wt_ref, b_ref, out_ref):
    out_ref[...] = (
        jnp.dot(nf_ref[...], wt_ref[...], preferred_element_type=jnp.float32)
        + b_ref[...]
    )


def _me_kernel(hs_ref, ef_ref, w1p_ref, b1p_ref, wct_ref, out_ref):
    f32 = jnp.float32
    EB = 8 * RME
    ehp = jnp.maximum(
        jnp.dot(ef_ref[...], w1p_ref[...], preferred_element_type=f32)
        + b1p_ref[...], 0.0)
    ehT = ehp.T.reshape(8, H, RME).transpose(1, 0, 2).reshape(H, EB)
    hsT = hs_ref[...].T.reshape(8, H, RME).transpose(1, 0, 2).reshape(H, EB)
    ehx = jnp.broadcast_to(ehT[None, :, :], (H, H, EB)).reshape(H * H, EB)
    hsx = jnp.broadcast_to(hsT[:, None, :], (H, H, EB)).reshape(H * H, EB)
    pt = jnp.concatenate([hsx * ehx, hsT], axis=0)
    meT = jnp.dot(wct_ref[...], pt, preferred_element_type=f32)
    mePT = meT.reshape(H, 8, RME).transpose(1, 0, 2).reshape(128, RME)
    out_ref[...] = mePT.T


def _gru_kernel(h_ref, m2_ref, wih_ref, whh_ref, bih_ref, bhh_ref, out_ref):
    m = m2_ref[0] + m2_ref[1]
    gi = jnp.dot(m, wih_ref[...], preferred_element_type=jnp.float32) + bih_ref[...]
    gh = (
        jnp.dot(h_ref[...], whh_ref[...], preferred_element_type=jnp.float32)
        + bhh_ref[...]
    )
    r = jax.nn.sigmoid(gi[:, 0:H] + gh[:, 0:H])
    z = jax.nn.sigmoid(gi[:, H:2 * H] + gh[:, H:2 * H])
    n = jnp.tanh(gi[:, 2 * H:] + r * gh[:, 2 * H:])
    out_ref[...] = (1.0 - z) * n + z * h_ref[...]


def _set2set_kernel(h_ref, b2d_ref, bt_ref, wiht_ref, whht_ref, bi_ref, bh_ref,
                    woutt_ref, bout_ref, out_ref):
    h = h_ref[...]
    onehot = jnp.where(
        b2d_ref[...] == lax.broadcasted_iota(jnp.int32, (N, B), 1), 1.0, 0.0
    )
    onehot_t = jnp.where(
        bt_ref[...] == lax.broadcasted_iota(jnp.int32, (B, N), 0), 1.0, 0.0
    )
    qstar = jnp.zeros((B, 2 * H), jnp.float32)
    hx = jnp.zeros((B, H), jnp.float32)
    cx = jnp.zeros((B, H), jnp.float32)
    for _ in range(STEPS):
        gates = (
            jnp.dot(qstar, wiht_ref[...], preferred_element_type=jnp.float32)
            + bi_ref[...]
            + jnp.dot(hx, whht_ref[...], preferred_element_type=jnp.float32)
            + bh_ref[...]
        )
        ig = jax.nn.sigmoid(gates[:, 0:H])
        fg = jax.nn.sigmoid(gates[:, H:2 * H])
        gg = jnp.tanh(gates[:, 2 * H:3 * H])
        og = jax.nn.sigmoid(gates[:, 3 * H:])
        cx = fg * cx + ig * gg
        hx = og * jnp.tanh(cx)
        qb = jnp.dot(onehot, hx, preferred_element_type=jnp.float32)
        e = jnp.sum(h * qb, axis=1, keepdims=True)
        em = jnp.where(onehot > 0.0, e, -1e30)
        maxv = jnp.max(em, axis=0, keepdims=True)
        maxn = jnp.sum(onehot * maxv, axis=1, keepdims=True)
        expv = jnp.exp(e - maxn)
        denom = jnp.dot(onehot_t, expv, preferred_element_type=jnp.float32)
        denn = jnp.dot(onehot, denom, preferred_element_type=jnp.float32)
        a = expv / denn
        rvec = jnp.dot(onehot_t, a * h, preferred_element_type=jnp.float32)
        qstar = jnp.concatenate([hx, rvec], axis=1)
    out_ref[...] = (
        jnp.dot(hx, woutt_ref[...], preferred_element_type=jnp.float32)
        + bout_ref[...]
    )


def _full(shape):
    return pl.BlockSpec(shape, lambda *_: tuple(0 for _ in shape))


def kernel(node_features, edge_features, Esrc, Etgt, batch,
           W_in, b_in, ee_W1, ee_b1, ee_W2, ee_b2,
           gru_Wih, gru_Whh, gru_bih, gru_bhh,
           lstm_Wih, lstm_Whh, lstm_bih, lstm_bhh,
           W_out, b_out):
    f32 = jnp.float32
    es3 = jnp.pad(
        Esrc.astype(jnp.int32).reshape(NW, EW), ((0, 0), (0, PAD))
    ).reshape(NW, CHUNKS, CH)
    et3 = jnp.pad(
        Etgt.astype(jnp.int32).reshape(NW, EW), ((0, 0), (0, PAD)),
        constant_values=N,
    ).reshape(NW, CHUNKS, CH)

    wct = jnp.concatenate(
        [ee_W2.reshape(H, H * H), ee_b2.reshape(H, H)], axis=1
    )
    w1p = jnp.kron(jnp.eye(8, dtype=f32), ee_W1.T)
    b1p = jnp.tile(ee_b1, 8).reshape(1, 128)

    ef128 = jnp.pad(edge_features.reshape(E // 8, 128),
                    ((0, (EP - E) // 8), (0, 0)))

    h = pl.pallas_call(
        _proj_kernel,
        out_shape=jax.ShapeDtypeStruct((N, H), f32),
        in_specs=[_full((N, F_NODE)), _full((F_NODE, H)), _full((1, H))],
        out_specs=_full((N, H)),
    )(node_features, W_in.T, b_in.reshape(1, H))

    me_call = pl.pallas_call(
        _me_kernel,
        out_shape=jax.ShapeDtypeStruct((RP, 128), f32),
        grid=(RP // RME,),
        in_specs=[
            pl.BlockSpec((RME, 128), lambda i: (i, 0)),
            pl.BlockSpec((RME, 128), lambda i: (i, 0)),
            _full((128, 128)),
            _full((1, 128)),
            _full((H, H * H + H)),
        ],
        out_specs=pl.BlockSpec((RME, 128), lambda i: (i, 0)),
    )

    gru_call = pl.pallas_call(
        _gru_kernel,
        out_shape=jax.ShapeDtypeStruct((N, H), f32),
        in_specs=[
            _full((N, H)),
            _full((NC, N, H)),
            _full((H, 3 * H)),
            _full((H, 3 * H)),
            _full((1, 3 * H)),
            _full((1, 3 * H)),
        ],
        out_specs=_full((N, H)),
    )
    wih_t = gru_Wih.T
    whh_t = gru_Whh.T
    bih2 = gru_bih.reshape(1, 3 * H)
    bhh2 = gru_bhh.reshape(1, 3 * H)

    for _ in range(T):
        hs = _sc_gather(h, es3)
        me128 = me_call(hs.reshape(RP, 128), ef128, w1p, b1p, wct)
        m2 = _sc_scatter(me128.reshape(EP, H), et3)
        h = gru_call(h, m2, wih_t, whh_t, bih2, bhh2)

    out = pl.pallas_call(
        _set2set_kernel,
        out_shape=jax.ShapeDtypeStruct((B, 1), f32),
        in_specs=[
            _full((N, H)),
            _full((N, 1)),
            _full((1, N)),
            _full((2 * H, 4 * H)),
            _full((H, 4 * H)),
            _full((1, 4 * H)),
            _full((1, 4 * H)),
            _full((H, 1)),
            _full((1, 1)),
        ],
        out_specs=_full((B, 1)),
    )(
        h,
        batch.astype(jnp.int32).reshape(N, 1),
        batch.astype(jnp.int32).reshape(1, N),
        lstm_Wih.T,
        lstm_Whh.T,
        lstm_bih.reshape(1, 4 * H),
        lstm_bhh.reshape(1, 4 * H),
        W_out.T,
        b_out.reshape(1, 1),
    )
    return out


# speedup vs baseline: 12.8721x; 1.0799x over previous
import functools

import jax
import jax.numpy as jnp
import numpy as np
from jax import lax
from jax.experimental import pallas as pl
from jax.experimental.pallas import tpu as pltpu
from jax.experimental.pallas import tpu_sc as plsc

N = 10000
E = 160000
F_NODE = 128
F_EDGE = 16
H = 16
T = 3
STEPS = 12
B = 64

NC = 2
NS = 16
NW = NC * NS
EW = E // NW
CH = 128
CHUNKS = 40
EWP = CHUNKS * CH
PAD = EWP - EW
ACC_N = 10240
NPS = N // NS
NPZ = ACC_N // NS
EP = 163840
RP = EP // 8
RME = 512


def _sc_mesh():
    return plsc.VectorSubcoreMesh(
        core_axis_name="c", subcore_axis_name="s", num_cores=NC, num_subcores=NS
    )


def _gather_body(h_hbm, idx_hbm, out_hbm, idx_v, rows_v, sem):
    wid = lax.axis_index("s") * NC + lax.axis_index("c")
    pltpu.sync_copy(idx_hbm.at[wid], idx_v)
    descs = [
        pltpu.async_copy(
            h_hbm.at[idx_v.at[j]], rows_v.at[pl.ds(j * CH, CH)], sem
        )
        for j in range(CHUNKS)
    ]
    for d in descs:
        d.wait()
    base = pl.multiple_of(wid * EW, 8)
    pltpu.sync_copy(rows_v.at[pl.ds(0, EW)], out_hbm.at[pl.ds(base, EW)])


def _sc_gather(h, es3):
    k = pl.kernel(
        _gather_body,
        out_type=jax.ShapeDtypeStruct((EP, H), jnp.float32),
        mesh=_sc_mesh(),
        scratch_types=[
            pltpu.VMEM((CHUNKS, CH), jnp.int32),
            pltpu.VMEM((EWP, H), jnp.float32),
            pltpu.SemaphoreType.DMA,
        ],
        compiler_params=pltpu.CompilerParams(use_tc_tiling_on_sc=False),
    )
    return k(h, es3)


def _scatter_body(me_hbm, idx_hbm, out_hbm, idx_v, rows_v, buf_v, acc_sh, sem):
    cid = lax.axis_index("c")
    sid = lax.axis_index("s")
    wid = sid * NC + cid

    def zbody(i, _):
        buf_v[i, :] = jnp.zeros((H,), jnp.float32)
        return 0

    lax.fori_loop(0, NPZ, zbody, 0)
    pltpu.sync_copy(buf_v, acc_sh.at[pl.ds(sid * NPZ, NPZ)])
    pltpu.sync_copy(idx_hbm.at[wid], idx_v)
    base = pl.multiple_of(wid * EW, 8)
    pltpu.sync_copy(me_hbm.at[pl.ds(base, EW)], rows_v.at[pl.ds(0, EW)])
    plsc.subcore_barrier()
    descs = [
        pltpu.async_copy(
            rows_v.at[pl.ds(j * CH, CH)], acc_sh.at[idx_v.at[j]], sem, add=True
        )
        for j in range(CHUNKS)
    ]
    for d in descs:
        d.wait()
    plsc.subcore_barrier()

    pltpu.sync_copy(acc_sh.at[pl.ds(sid * NPS, NPS)], buf_v.at[pl.ds(0, NPS)])
    pltpu.sync_copy(buf_v.at[pl.ds(0, NPS)], out_hbm.at[cid].at[pl.ds(sid * NPS, NPS)])


def _sc_scatter(m_e, et3):
    k = pl.kernel(
        _scatter_body,
        out_type=jax.ShapeDtypeStruct((NC, N, H), jnp.float32),
        mesh=_sc_mesh(),
        scratch_types=[
            pltpu.VMEM((CHUNKS, CH), jnp.int32),
            pltpu.VMEM((EWP, H), jnp.float32),
            pltpu.VMEM((NPZ, H), jnp.float32),
            pltpu.VMEM_SHARED((ACC_N, H), jnp.float32),
            pltpu.SemaphoreType.DMA,
        ],
        compiler_params=pltpu.CompilerParams(use_tc_tiling_on_sc=False),
    )
    return k(m_e, et3)


def _proj_kernel(nf_ref, wt_ref, b_ref, out_ref):
    out_ref[...] = (
        jnp.dot(nf_ref[...], wt_ref[...], preferred_element_type=jnp.float32)
        + b_ref[...]
    )


def _me_kernel(hs_ref, ef_ref, w1p_ref, b1p_ref, wct_ref, out_ref):
    f32 = jnp.float32
    EB = 8 * RME
    ehp = jnp.maximum(
        jnp.dot(ef_ref[...], w1p_ref[...], preferred_element_type=f32)
        + b1p_ref[...], 0.0)
    ehT = ehp.T.reshape(8, H, RME).transpose(1, 0, 2).reshape(H, EB)
    hsT = hs_ref[...].T.reshape(8, H, RME).transpose(1, 0, 2).reshape(H, EB)
    ehx = jnp.broadcast_to(ehT[None, :, :], (H, H, EB)).reshape(H * H, EB)
    hsx = jnp.broadcast_to(hsT[:, None, :], (H, H, EB)).reshape(H * H, EB)
    pt = jnp.concatenate([hsx * ehx, hsT], axis=0)
    meT = jnp.dot(wct_ref[...], pt, preferred_element_type=f32)
    mePT = meT.reshape(H, 8, RME).transpose(1, 0, 2).reshape(128, RME)
    out_ref[...] = mePT.T


def _gru_kernel(h_ref, m2_ref, wih_ref, whh_ref, bih_ref, bhh_ref, out_ref):
    m = m2_ref[0] + m2_ref[1]
    gi = jnp.dot(m, wih_ref[...], preferred_element_type=jnp.float32) + bih_ref[...]
    gh = (
        jnp.dot(h_ref[...], whh_ref[...], preferred_element_type=jnp.float32)
        + bhh_ref[...]
    )
    r = jax.nn.sigmoid(gi[:, 0:H] + gh[:, 0:H])
    z = jax.nn.sigmoid(gi[:, H:2 * H] + gh[:, H:2 * H])
    n = jnp.tanh(gi[:, 2 * H:] + r * gh[:, 2 * H:])
    out_ref[...] = (1.0 - z) * n + z * h_ref[...]


def _set2set_kernel(h_ref, b2d_ref, bt_ref, wiht_ref, whht_ref, bi_ref, bh_ref,
                    woutt_ref, bout_ref, out_ref):
    f32 = jnp.float32
    hT = h_ref[...].T
    onehot_n = jnp.where(
        b2d_ref[...] == lax.broadcasted_iota(jnp.int32, (N, B), 1), 1.0, 0.0
    )
    onehot_t = jnp.where(
        bt_ref[...] == lax.broadcasted_iota(jnp.int32, (B, N), 0), 1.0, 0.0
    )
    qstar = jnp.zeros((B, 2 * H), f32)
    hx = jnp.zeros((B, H), f32)
    cx = jnp.zeros((B, H), f32)
    for _ in range(STEPS):
        gates = (
            jnp.dot(qstar, wiht_ref[...], preferred_element_type=f32)
            + bi_ref[...]
            + jnp.dot(hx, whht_ref[...], preferred_element_type=f32)
            + bh_ref[...]
        )
        ig = jax.nn.sigmoid(gates[:, 0:H])
        fg = jax.nn.sigmoid(gates[:, H:2 * H])
        gg = jnp.tanh(gates[:, 2 * H:3 * H])
        og = jax.nn.sigmoid(gates[:, 3 * H:])
        cx = fg * cx + ig * gg
        hx = og * jnp.tanh(cx)
        qbT = jnp.dot(hx.T, onehot_t, preferred_element_type=f32)
        eT = jnp.sum(hT * qbT, axis=0, keepdims=True)
        emT = jnp.where(onehot_t > 0.0, eT, -1e30)
        maxv = jnp.max(emT, axis=1, keepdims=True)
        maxnT = jnp.dot(maxv.T, onehot_t, preferred_element_type=f32)
        expvT = jnp.exp(eT - maxnT)
        denom = jnp.sum(onehot_t * expvT, axis=1, keepdims=True)
        dennT = jnp.dot(denom.T, onehot_t, preferred_element_type=f32)
        aT = expvT / jnp.maximum(dennT, 1e-30)
        rvecT = jnp.dot(hT * aT, onehot_n, preferred_element_type=f32)
        qstar = jnp.concatenate([hx, rvecT.T], axis=1)
    out_ref[...] = (
        jnp.dot(hx, woutt_ref[...], preferred_element_type=f32)
        + bout_ref[...]
    )


def _full(shape):
    return pl.BlockSpec(shape, lambda *_: tuple(0 for _ in shape))


def kernel(node_features, edge_features, Esrc, Etgt, batch,
           W_in, b_in, ee_W1, ee_b1, ee_W2, ee_b2,
           gru_Wih, gru_Whh, gru_bih, gru_bhh,
           lstm_Wih, lstm_Whh, lstm_bih, lstm_bhh,
           W_out, b_out):
    f32 = jnp.float32
    es3 = jnp.pad(
        Esrc.astype(jnp.int32).reshape(NW, EW), ((0, 0), (0, PAD))
    ).reshape(NW, CHUNKS, CH)
    et3 = jnp.pad(
        Etgt.astype(jnp.int32).reshape(NW, EW), ((0, 0), (0, PAD)),
        constant_values=N,
    ).reshape(NW, CHUNKS, CH)

    wct = jnp.concatenate(
        [ee_W2.reshape(H, H * H), ee_b2.reshape(H, H)], axis=1
    )
    w1p = jnp.kron(jnp.eye(8, dtype=f32), ee_W1.T)
    b1p = jnp.tile(ee_b1, 8).reshape(1, 128)

    ef128 = jnp.pad(edge_features.reshape(E // 8, 128),
                    ((0, (EP - E) // 8), (0, 0)))

    h = pl.pallas_call(
        _proj_kernel,
        out_shape=jax.ShapeDtypeStruct((N, H), f32),
        in_specs=[_full((N, F_NODE)), _full((F_NODE, H)), _full((1, H))],
        out_specs=_full((N, H)),
    )(node_features, W_in.T, b_in.reshape(1, H))

    me_call = pl.pallas_call(
        _me_kernel,
        out_shape=jax.ShapeDtypeStruct((RP, 128), f32),
        grid=(RP // RME,),
        in_specs=[
            pl.BlockSpec((RME, 128), lambda i: (i, 0)),
            pl.BlockSpec((RME, 128), lambda i: (i, 0)),
            _full((128, 128)),
            _full((1, 128)),
            _full((H, H * H + H)),
        ],
        out_specs=pl.BlockSpec((RME, 128), lambda i: (i, 0)),
    )

    gru_call = pl.pallas_call(
        _gru_kernel,
        out_shape=jax.ShapeDtypeStruct((N, H), f32),
        in_specs=[
            _full((N, H)),
            _full((NC, N, H)),
            _full((H, 3 * H)),
            _full((H, 3 * H)),
            _full((1, 3 * H)),
            _full((1, 3 * H)),
        ],
        out_specs=_full((N, H)),
    )
    wih_t = gru_Wih.T
    whh_t = gru_Whh.T
    bih2 = gru_bih.reshape(1, 3 * H)
    bhh2 = gru_bhh.reshape(1, 3 * H)

    for _ in range(T):
        hs = _sc_gather(h, es3)
        me128 = me_call(hs.reshape(RP, 128), ef128, w1p, b1p, wct)
        m2 = _sc_scatter(me128.reshape(EP, H), et3)
        h = gru_call(h, m2, wih_t, whh_t, bih2, bhh2)

    out = pl.pallas_call(
        _set2set_kernel,
        out_shape=jax.ShapeDtypeStruct((B, 1), f32),
        in_specs=[
            _full((N, H)),
            _full((N, 1)),
            _full((1, N)),
            _full((2 * H, 4 * H)),
            _full((H, 4 * H)),
            _full((1, 4 * H)),
            _full((1, 4 * H)),
            _full((H, 1)),
            _full((1, 1)),
        ],
        out_specs=_full((B, 1)),
    )(
        h,
        batch.astype(jnp.int32).reshape(N, 1),
        batch.astype(jnp.int32).reshape(1, N),
        lstm_Wih.T,
        lstm_Whh.T,
        lstm_bih.reshape(1, 4 * H),
        lstm_bhh.reshape(1, 4 * H),
        W_out.T,
        b_out.reshape(1, 1),
    )
    return out
